# baseline (speedup 1.0000x reference)
import functools
import math

import jax
import jax.numpy as jnp
from jax.experimental import pallas as pl
from jax.experimental.pallas import tpu as pltpu

_GAMMA = 0.1
_ALPHA = 0.3
_EPS = 1e-8
_BIG = 100000000.0
_C = 1.0 / _GAMMA
_ABIG = -_BIG * _C
_F = 8
_CH = 128


def _loss_kernel(pe_ref, trp_ref, pT_ref, tT_ref, out_ref, a1_s, shp_s, *, n):
    bb = pe_ref.shape[1]
    w = n + 8
    p = pT_ref[...]
    t = tT_ref[...]

    inv_n = jnp.float32(1.0 / n)
    pm = jnp.sum(p, axis=0, keepdims=True) * inv_n
    tm = jnp.sum(t, axis=0, keepdims=True) * inv_n
    pc = p - pm
    tc = t - tm
    dot = jnp.sum(pc * tc, axis=0, keepdims=True)
    nrm = jnp.sqrt(jnp.sum(pc * pc, axis=0, keepdims=True)) * jnp.sqrt(
        jnp.sum(tc * tc, axis=0, keepdims=True))
    corr = dot / jnp.maximum(nrm, _EPS)
    pear = jnp.sum(corr)

    abig = jnp.float32(_ABIG)
    rows = jax.lax.broadcasted_iota(jnp.int32, (w + 8, bb), 0)
    a1_s[...] = jnp.full((w + 8, bb), abig)
    shp_s[...] = jnp.where(rows == 9, jnp.float32(0.0), abig)

    t0 = ((w - 8) // _CH) * _CH
    nfull = t0 // _CH
    abig_row = jnp.full((1, bb), abig)

    def substeps(pcur, ppsh, pe_c, obase, wlen):
        for j in range(1, _F + 1):
            xs = jnp.concatenate([abig_row, pcur[:-1]], axis=0)
            am = jnp.maximum(jnp.maximum(xs, pcur), ppsh)
            z = (jnp.exp(xs - am) + jnp.exp(pcur - am)
                 + jnp.exp(ppsh - am))
            tr = trp_ref[pl.ds(obase - (j - 1), wlen), :]
            dv = pe_c - tr
            newp = am + jnp.log(z) - dv * dv
            ppsh = xs
            pcur = newp
        return pcur, ppsh

    def group(gi, carry):
        d0 = 2 + _F * gi
        sbase = 2 * n + 9 - d0

        def top_chunk():
            pcur = a1_s[t0:w + 8]
            ppsh = shp_s[t0:w + 8]
            pe_c = pe_ref[t0:w + 8]
            pcur, ppsh = substeps(pcur, ppsh, pe_c, sbase + t0 - 8,
                                  w + 8 - t0)
            a1_s[t0 + 8:w + 8] = pcur[8:]
            shp_s[t0 + 8:w + 8] = ppsh[8:]

        pl.when(t0 <= d0 + 6)(top_chunk)

        def chunk(ti, c, hi):
            ci = hi - 1 - ti
            base = pl.multiple_of(ci * _CH, 8)
            pcur = a1_s[pl.ds(base, _CH + 8), :]
            ppsh = shp_s[pl.ds(base, _CH + 8), :]
            pe_c = pe_ref[pl.ds(base, _CH + 8), :]
            pcur, ppsh = substeps(pcur, ppsh, pe_c,
                                  sbase + ci * _CH - 8, _CH + 8)
            st = pl.multiple_of(ci * _CH + 8, 8)
            a1_s[pl.ds(st, _CH), :] = pcur[8:]
            shp_s[pl.ds(st, _CH), :] = ppsh[8:]
            return c

        if nfull > 0:
            hi = jnp.minimum(nfull, (d0 + 6) // _CH + 1)
            lo = jnp.maximum(0, (d0 - n - 1) // _CH)
            jax.lax.fori_loop(0, hi - lo,
                              functools.partial(chunk, hi=hi), jnp.int32(0))
        return carry

    jax.lax.fori_loop(0, (2 * n) // _F, group, jnp.int32(0))
    dtw = jnp.sum(shp_s[n + 9:n + 10, :])
    out_ref[0:1, :] = jnp.full((1, bb), dtw)
    out_ref[1:2, :] = jnp.full((1, bb), pear)
    out_ref[2:8, :] = jnp.zeros((6, bb))


def _forward(pred, target, interpret=False):
    b, n = pred.shape
    bb = 128 if b % 128 == 0 else b
    grid = b // bb
    f32 = jnp.float32
    s = jnp.float32(math.sqrt(_C))
    predt = pred.T.astype(f32)
    tt = target.T.astype(f32)
    pst = predt * s
    trevt = jnp.flip(target, axis=1).T.astype(f32) * s
    pe = jnp.concatenate(
        [jnp.zeros((8, b), f32), pst[:1], pst, jnp.zeros((7, b), f32)],
        axis=0)
    trp = jnp.concatenate(
        [jnp.zeros((n + 9, b), f32), trevt, jnp.zeros((n + 7, b), f32)],
        axis=0)

    out = pl.pallas_call(
        functools.partial(_loss_kernel, n=n),
        grid=(grid,),
        in_specs=[
            pl.BlockSpec((n + 16, bb), lambda gi: (0, gi)),
            pl.BlockSpec((3 * n + 16, bb), lambda gi: (0, gi)),
            pl.BlockSpec((n, bb), lambda gi: (0, gi)),
            pl.BlockSpec((n, bb), lambda gi: (0, gi)),
        ],
        out_specs=pl.BlockSpec((8, bb), lambda gi: (gi, 0)),
        out_shape=jax.ShapeDtypeStruct((8 * grid, bb), f32),
        scratch_shapes=[
            pltpu.VMEM((n + 16, bb), f32),
            pltpu.VMEM((n + 16, bb), f32),
        ],
        compiler_params=pltpu.CompilerParams(
            dimension_semantics=("parallel",)),
        interpret=interpret,
    )(pe, trp, predt, tt)

    outr = out.reshape(grid, 8, bb)
    dtw_mean = -jnp.sum(outr[:, 0, 0]) / (b * _C)
    pear_mean = jnp.sum(outr[:, 1, 0]) / b
    return _ALPHA * dtw_mean + (1.0 - _ALPHA) * (1.0 - pear_mean)


@jax.jit
def _forward_compiled(pred, target):
    return _forward(pred, target)


def kernel(pred, target):
    return _forward_compiled(pred, target)


# speedup vs baseline: 1.0071x; 1.0071x over previous
import functools
import math

import jax
import jax.numpy as jnp
from jax.experimental import pallas as pl
from jax.experimental.pallas import tpu as pltpu

_GAMMA = 0.1
_ALPHA = 0.3
_EPS = 1e-8
_BIG = 100000000.0
_C = 1.0 / _GAMMA
_ABIG = -_BIG * _C
_F = 8


def _loss_kernel(pe_ref, trp_ref, pT_ref, tT_ref, out_ref, a1_s, shp_s, *, n):
    bb = pe_ref.shape[1]
    w = n + 8
    p = pT_ref[...]
    t = tT_ref[...]

    inv_n = jnp.float32(1.0 / n)
    pm = jnp.sum(p, axis=0, keepdims=True) * inv_n
    tm = jnp.sum(t, axis=0, keepdims=True) * inv_n
    pc = p - pm
    tc = t - tm
    dot = jnp.sum(pc * tc, axis=0, keepdims=True)
    nrm = jnp.sqrt(jnp.sum(pc * pc, axis=0, keepdims=True)) * jnp.sqrt(
        jnp.sum(tc * tc, axis=0, keepdims=True))
    corr = dot / jnp.maximum(nrm, _EPS)
    pear = jnp.sum(corr)

    abig = jnp.float32(_ABIG)
    rows = jax.lax.broadcasted_iota(jnp.int32, (w, bb), 0)
    a1_s[...] = jnp.full((w, bb), abig)
    shp_s[...] = jnp.where(rows == 1, jnp.float32(0.0), abig)

    t0 = ((w - 8) // 128) * 128
    bounds = [(t0, w)] + [(r, r + 128) for r in range(t0 - 128, -1, -128)]
    fill8 = jnp.full((8, bb), abig)
    zfill8 = jnp.zeros((8, bb), jnp.float32)
    abig_row = jnp.full((1, bb), abig)

    def group(gi, carry):
        d0 = 2 + _F * gi
        sbase = 2 * n + 9 - d0
        def chunk_group(pair):
            for r0, r1 in pair:
                chunk_once(r0, r1)

        def chunk_once(r0, r1):
            c0 = r0 - 8

            def chunk_body(r0=r0, r1=r1, c0=c0):
                if r0 == 0:
                    pcur = jnp.concatenate([fill8, a1_s[0:r1]], axis=0)
                    ppsh = jnp.concatenate([fill8, shp_s[0:r1]], axis=0)
                    pe_c = jnp.concatenate([zfill8, pe_ref[0:r1]], axis=0)
                else:
                    pcur = a1_s[c0:r1]
                    ppsh = shp_s[c0:r1]
                    pe_c = pe_ref[c0:r1]
                wlen = r1 - c0
                obase = sbase + c0
                for j in range(1, _F + 1):
                    xs = jnp.concatenate([abig_row, pcur[:-1]], axis=0)
                    am = jnp.maximum(jnp.maximum(xs, pcur), ppsh)
                    z = (jnp.exp(xs - am) + jnp.exp(pcur - am)
                         + jnp.exp(ppsh - am))
                    tr = trp_ref[pl.ds(obase - (j - 1), wlen), :]
                    dv = pe_c - tr
                    newp = am + jnp.log(z) - dv * dv
                    ppsh = xs
                    pcur = newp
                a1_s[r0:r1] = pcur[8:]
                shp_s[r0:r1] = ppsh[8:]

            chunk_body()

        for i in range(len(bounds)):
            grp = bounds[i:i + 1]
            lo_r0 = grp[-1][0]
            hi_r1 = grp[0][1]
            pl.when((lo_r0 <= d0 + 6) & (d0 <= n + hi_r1))(
                functools.partial(chunk_group, grp))
        return carry

    jax.lax.fori_loop(0, (2 * n) // _F, group, jnp.int32(0))
    dtw = jnp.sum(shp_s[n + 1:n + 2, :])
    out_ref[0:1, :] = jnp.full((1, bb), dtw)
    out_ref[1:2, :] = jnp.full((1, bb), pear)
    out_ref[2:8, :] = jnp.zeros((6, bb))


def _forward(pred, target, interpret=False):
    b, n = pred.shape
    bb = 128 if b % 128 == 0 else b
    grid = b // bb
    f32 = jnp.float32
    s = jnp.float32(math.sqrt(_C))
    predt = pred.T.astype(f32)
    tt = target.T.astype(f32)
    pst = predt * s
    trevt = jnp.flip(target, axis=1).T.astype(f32) * s
    zc = jnp.zeros((7, b), f32)
    pe = jnp.concatenate([pst[:1], pst, zc], axis=0)
    trp = jnp.concatenate(
        [jnp.zeros((n + 9, b), f32), trevt, jnp.zeros((n + 7, b), f32)],
        axis=0)

    out = pl.pallas_call(
        functools.partial(_loss_kernel, n=n),
        grid=(grid,),
        in_specs=[
            pl.BlockSpec((n + 8, bb), lambda gi: (0, gi)),
            pl.BlockSpec((3 * n + 16, bb), lambda gi: (0, gi)),
            pl.BlockSpec((n, bb), lambda gi: (0, gi)),
            pl.BlockSpec((n, bb), lambda gi: (0, gi)),
        ],
        out_specs=pl.BlockSpec((8, bb), lambda gi: (gi, 0)),
        out_shape=jax.ShapeDtypeStruct((8 * grid, bb), f32),
        scratch_shapes=[
            pltpu.VMEM((n + 8, bb), f32),
            pltpu.VMEM((n + 8, bb), f32),
        ],
        compiler_params=pltpu.CompilerParams(
            dimension_semantics=("parallel",)),
        interpret=interpret,
    )(pe, trp, predt, tt)

    outr = out.reshape(grid, 8, bb)
    dtw_mean = -jnp.sum(outr[:, 0, 0]) / (b * _C)
    pear_mean = jnp.sum(outr[:, 1, 0]) / b
    return _ALPHA * dtw_mean + (1.0 - _ALPHA) * (1.0 - pear_mean)


@jax.jit
def _forward_compiled(pred, target):
    return _forward(pred, target)


def kernel(pred, target):
    return _forward_compiled(pred, target)
